# baseline (speedup 1.0000x reference)
import jax
import jax.numpy as jnp
from jax import lax
from jax.experimental import pallas as pl
from jax.experimental.pallas import tpu as pltpu
from jax.experimental.pallas import tpu_sc as plsc

_FACTORS = 64
_LANES = 16
_NUM_WORKERS = 32
_BATCH = 16384
_BPW = _BATCH // _NUM_WORKERS


def _mf_body(users_hbm, items_hbm, uhi_hbm, ihi_hbm, l_hbm, r_hbm,
             lb_hbm, rb_hbm, out_hbm,
             uidx_v, iidx_v, uhi_v, ihi_v, urows_v, irows_v, ub_v, rb_v,
             out_v, sem):
    wid = lax.axis_index("s") * 2 + lax.axis_index("c")
    base = wid * _BPW

    pltpu.sync_copy(users_hbm.at[wid], uidx_v)
    pltpu.sync_copy(items_hbm.at[wid], iidx_v)
    pltpu.sync_copy(uhi_hbm.at[wid], uhi_v)
    pltpu.sync_copy(ihi_hbm.at[wid], ihi_v)

    c0 = pltpu.async_copy(l_hbm.at[uidx_v], urows_v, sem)
    c1 = pltpu.async_copy(r_hbm.at[iidx_v], irows_v, sem)
    c2 = pltpu.async_copy(lb_hbm.at[uhi_v], ub_v, sem)
    c3 = pltpu.async_copy(rb_hbm.at[ihi_v], rb_v, sem)
    c0.wait()
    c1.wait()
    c2.wait()
    c3.wait()

    def group(g, carry):
        rows = g * _LANES + lax.iota(jnp.int32, _LANES)
        accs = [jnp.zeros((_LANES,), jnp.float32) for _ in range(4)]
        for k in range(_FACTORS):
            col = jnp.full((_LANES,), k, jnp.int32)
            uk = plsc.load_gather(urows_v, [rows, col])
            ik = plsc.load_gather(irows_v, [rows, col])
            accs[k % 4] = accs[k % 4] + uk * ik
        dot = (accs[0] + accs[1]) + (accs[2] + accs[3])
        ulo = uidx_v[pl.ds(g * _LANES, _LANES)] & 15
        ilo = iidx_v[pl.ds(g * _LANES, _LANES)] & 15
        ub = plsc.load_gather(ub_v, [rows, ulo])
        rb = plsc.load_gather(rb_v, [rows, ilo])
        out_v[pl.ds(g * _LANES, _LANES)] = dot + ub + rb
        return carry

    lax.fori_loop(0, _BPW // _LANES, group, 0)
    pltpu.sync_copy(out_v, out_hbm.at[pl.ds(base, _BPW)])


def kernel(minibatch, L, R, L_bias, R_bias):
    users = minibatch[:, 0].reshape(_NUM_WORKERS, _BPW)
    items = minibatch[:, 1].reshape(_NUM_WORKERS, _BPW)
    lb16 = L_bias.reshape(L_bias.shape[0] // _LANES, _LANES)
    rb16 = R_bias.reshape(R_bias.shape[0] // _LANES, _LANES)
    mesh = plsc.VectorSubcoreMesh(
        core_axis_name="c", subcore_axis_name="s")
    f = pl.kernel(
        _mf_body,
        out_type=jax.ShapeDtypeStruct((_BATCH,), jnp.float32),
        mesh=mesh,
        scratch_types=[
            pltpu.VMEM((_BPW,), jnp.int32),
            pltpu.VMEM((_BPW,), jnp.int32),
            pltpu.VMEM((_BPW,), jnp.int32),
            pltpu.VMEM((_BPW,), jnp.int32),
            pltpu.VMEM((_BPW, _FACTORS), jnp.float32),
            pltpu.VMEM((_BPW, _FACTORS), jnp.float32),
            pltpu.VMEM((_BPW, _LANES), jnp.float32),
            pltpu.VMEM((_BPW, _LANES), jnp.float32),
            pltpu.VMEM((_BPW,), jnp.float32),
            pltpu.SemaphoreType.DMA,
        ],
        compiler_params=pltpu.CompilerParams(
            needs_layout_passes=False, use_tc_tiling_on_sc=False
        ),
    )
    return f(users, items, users >> 4, items >> 4, L, R, lb16, rb16)
